# baseline (speedup 1.0000x reference)
import functools

import jax
import jax.numpy as jnp
from jax import lax
from jax.experimental import pallas as pl
from jax.experimental.pallas import tpu as pltpu
from jax.experimental.pallas import tpu_sc as plsc

V = 1000000
D = 64
R = 16
N = 1024 * 200

NC = 2
NS = 16
NW = NC * NS
B_PER_W = N // NW
CHUNK = 320
NCHUNKS = B_PER_W // CHUNK
ACHUNK = 640
ANCHUNKS = B_PER_W // ACHUNK


def _worker_id():
    return lax.axis_index("s") * NC + lax.axis_index("c")


@functools.cache
def _sc_kernels():
    mesh = plsc.VectorSubcoreMesh(core_axis_name="c", subcore_axis_name="s")

    @functools.partial(
        pl.kernel,
        out_type=jax.ShapeDtypeStruct((N, 2 * D), jnp.float32),
        mesh=mesh,
        compiler_params=pltpu.CompilerParams(use_tc_tiling_on_sc=False),
        scratch_types=[
            pltpu.VMEM((B_PER_W,), jnp.int32),
            pltpu.VMEM((CHUNK, 2 * D), jnp.float32),
            pltpu.VMEM((CHUNK, 2 * D), jnp.float32),
            pltpu.SemaphoreType.DMA,
            pltpu.SemaphoreType.DMA,
        ],
    )
    def base_gather(idxh_hbm, table_hbm, out_hbm, idx_v, b0, b1, sem0, sem1):
        base = _worker_id() * B_PER_W
        pltpu.sync_copy(idxh_hbm.at[pl.ds(base, B_PER_W)], idx_v)
        bufs = (b0, b1)
        sems = (sem0, sem1)
        cps = [None, None]

        def start(k):
            j = k % 2
            cps[j] = pltpu.async_copy(
                table_hbm.at[idx_v.at[pl.ds(k * CHUNK, CHUNK)]], bufs[j], sems[j])

        start(0)
        for k in range(NCHUNKS):
            if k + 1 < NCHUNKS:
                start(k + 1)
            j = k % 2
            cps[j].wait()
            pltpu.sync_copy(bufs[j], out_hbm.at[pl.ds(base + k * CHUNK, CHUNK)])

    @functools.partial(
        pl.kernel,
        out_type=jax.ShapeDtypeStruct((N * R,), jnp.float32),
        mesh=mesh,
        compiler_params=pltpu.CompilerParams(use_tc_tiling_on_sc=False),
        scratch_types=[
            pltpu.VMEM((B_PER_W * R,), jnp.int32),
            pltpu.VMEM((ACHUNK * R,), jnp.float32),
            pltpu.VMEM((ACHUNK * R,), jnp.float32),
            pltpu.SemaphoreType.DMA,
            pltpu.SemaphoreType.DMA,
        ],
    )
    def lora_gather(idx16_hbm, aflat_hbm, out_hbm, idx_v, b0, b1, sem0, sem1):
        base = _worker_id() * B_PER_W * R
        pltpu.sync_copy(idx16_hbm.at[pl.ds(base, B_PER_W * R)], idx_v)
        bufs = (b0, b1)
        sems = (sem0, sem1)
        cps = [None, None]
        n_el = ACHUNK * R

        def start(k):
            j = k % 2
            cps[j] = pltpu.async_copy(
                aflat_hbm.at[idx_v.at[pl.ds(k * n_el, n_el)]], bufs[j], sems[j])

        start(0)
        for k in range(ANCHUNKS):
            if k + 1 < ANCHUNKS:
                start(k + 1)
            j = k % 2
            cps[j].wait()
            pltpu.sync_copy(bufs[j], out_hbm.at[pl.ds(base + k * n_el, n_el)])

    return base_gather, lora_gather


_BN = 2048


def _fuse_body(ar_ref, rows_ref, par_ref, b_ref, out_ref):
    rows = rows_ref[...]
    left = rows[:, :D]
    right = rows[:, D:]
    base = jnp.where(par_ref[...] > 0.5, right, left)
    out_ref[...] = base + jnp.dot(
        ar_ref[...], b_ref[...].T, preferred_element_type=jnp.float32
    )


_fuse = pl.pallas_call(
    _fuse_body,
    grid=(N // _BN,),
    in_specs=[
        pl.BlockSpec((_BN, R), lambda i: (i, 0)),
        pl.BlockSpec((_BN, 2 * D), lambda i: (i, 0)),
        pl.BlockSpec((_BN, 1), lambda i: (i, 0)),
        pl.BlockSpec((D, R), lambda i: (0, 0)),
    ],
    out_specs=pl.BlockSpec((_BN, D), lambda i: (i, 0)),
    out_shape=jax.ShapeDtypeStruct((N, D), jnp.float32),
)


def kernel(x, base_weight, lora_A, lora_B):
    Bsz, Ssz = x.shape
    idx = x.reshape(-1)
    idxh = idx >> 1
    par = (idx & 1).astype(jnp.float32).reshape(N, 1)
    idx16 = (idx[:, None] + (jnp.arange(R, dtype=jnp.int32) * V)[None, :]).reshape(-1)
    table2 = base_weight.reshape(V // 2, 2 * D)
    aflat = lora_A.reshape(R * V)
    base_gather, lora_gather = _sc_kernels()
    rows128 = base_gather(idxh, table2)
    ar = lora_gather(idx16, aflat).reshape(N, R)
    out = _fuse(ar, rows128, par, lora_B)
    return out.reshape(Bsz, Ssz, D)
